# baseline (speedup 1.0000x reference)
import functools

import jax
import jax.numpy as jnp
from jax import lax
from jax.experimental import pallas as pl
from jax.experimental.pallas import tpu as pltpu
from jax.experimental.pallas import tpu_sc as plsc

N = 10000
NP = 10240
D = 128
E = 320000
NC = 2
NS = 16
NW = NC * NS
EPT = 10240
K = 128
NB = EPT // K
CH = 40
ROWS_PER_TILE = NP // NS
BLK = 1024
GRID = NP // BLK

_mesh = plsc.VectorSubcoreMesh(core_axis_name="c", subcore_axis_name="s")


@functools.partial(
    pl.kernel, mesh=_mesh,
    out_type=jax.ShapeDtypeStruct((NW, NP), jnp.float32),
    scratch_types=[
        pltpu.VMEM((NB, K), jnp.int32),
        pltpu.VMEM((NP,), jnp.float32),
    ],
    compiler_params=pltpu.CompilerParams(needs_layout_passes=False),
)
def _deg_kernel(dst_hbm, out_hbm, dst_v, hist_v):
    cid = lax.axis_index("c")
    sid = lax.axis_index("s")
    wid = cid * NS + sid
    pltpu.sync_copy(dst_hbm.at[cid, sid], dst_v)

    zv = jnp.zeros((16,), jnp.float32)

    def zbody(i, carry):
        hist_v[pl.ds(i * 16, 16)] = zv
        return carry

    lax.fori_loop(0, NP // 16, zbody, 0)

    ones = jnp.ones((16,), jnp.float32)

    def body(r, carry):
        for c in range(K // 16):
            idx = dst_v[r, pl.ds(c * 16, 16)]
            plsc.addupdate_scatter(hist_v, [idx], ones)
        return carry

    lax.fori_loop(0, NB, body, 0)
    pltpu.sync_copy(hist_v, out_hbm.at[wid])


@functools.partial(
    pl.kernel, mesh=_mesh,
    out_type=jax.ShapeDtypeStruct((NC, NP, D), jnp.float32),
    scratch_types=[
        pltpu.VMEM((CH, K), jnp.int32),
        pltpu.VMEM((CH, K), jnp.int32),
        pltpu.VMEM((K, D), jnp.float32),
        pltpu.VMEM((K, D), jnp.float32),
        pltpu.VMEM((K, D), jnp.float32),
        pltpu.VMEM((K, D), jnp.float32),
        pltpu.SemaphoreType.DMA,
        pltpu.SemaphoreType.DMA,
        pltpu.SemaphoreType.DMA,
        pltpu.SemaphoreType.DMA,
    ],
    compiler_params=pltpu.CompilerParams(needs_layout_passes=False),
)
def _scatter_kernel(g_hbm, src_hbm, dst_hbm, out_hbm, src_v, dst_v, buf_a,
                    buf_b, buf_c, buf_d, sem_a, sem_b, sem_c, sem_d):
    bufs = (buf_a, buf_b, buf_c, buf_d)
    sems = (sem_a, sem_b, sem_c, sem_d)
    cid = lax.axis_index("c")
    sid = lax.axis_index("s")

    for ch in range(NB // CH):
        pltpu.sync_copy(src_hbm.at[cid, sid, pl.ds(ch * CH, CH)], src_v)
        pltpu.sync_copy(dst_hbm.at[cid, sid, pl.ds(ch * CH, CH)], dst_v)
        for u in range(4):
            pltpu.async_copy(g_hbm.at[src_v.at[u]], bufs[u], sems[u])

        def body(j, carry):
            base = j * 4
            for u in range(4):
                b = base + u
                pltpu.make_async_copy(g_hbm.at[src_v.at[b]], bufs[u],
                                      sems[u]).wait()
                nxt = jnp.minimum(b + 4, CH - 1)
                pltpu.async_copy(g_hbm.at[src_v.at[nxt]], bufs[u], sems[u])
            return carry

        lax.fori_loop(0, CH // 4, body, 0)
        for u in range(4):
            pltpu.make_async_copy(g_hbm.at[src_v.at[0]], bufs[u],
                                  sems[u]).wait()
    pltpu.sync_copy(buf_a, out_hbm.at[cid, pl.ds(sid * K, K)])


def _dinv_from(hist_ref):
    ones = jnp.ones((NW, D), jnp.float32)
    deg = lax.dot_general(hist_ref[...], ones, (((0,), (0,)), ((), ())),
                          preferred_element_type=jnp.float32)
    return lax.rsqrt(deg + 1.0)


def _tc_pre(x_ref, w_ref, hist_ref, g_ref):
    dinv = _dinv_from(hist_ref)
    g_ref[...] = dinv * jnp.dot(x_ref[...], w_ref[...],
                                preferred_element_type=jnp.float32)


def _tc_mid(p_ref, g_ref, hist_ref, w_ref, b_ref, o_ref):
    dinv = _dinv_from(hist_ref)
    s = p_ref[0] + p_ref[1] + g_ref[...]
    h = jnp.maximum(dinv * s + b_ref[...], 0.0)
    o_ref[...] = dinv * jnp.dot(h, w_ref[...],
                                preferred_element_type=jnp.float32)


def _tc_post(p_ref, g_ref, hist_ref, b_ref, o_ref):
    dinv = _dinv_from(hist_ref)
    s = p_ref[0] + p_ref[1] + g_ref[...]
    o_ref[...] = dinv * s + b_ref[...]


_f32 = jnp.float32
_hist_spec = pl.BlockSpec((NW, BLK), lambda i: (0, i))
_row_spec = pl.BlockSpec((BLK, D), lambda i: (i, 0))
_p_spec = pl.BlockSpec((NC, BLK, D), lambda i: (0, i, 0))
_w_spec = pl.BlockSpec((D, D), lambda i: (0, 0))
_b_spec = pl.BlockSpec((1, D), lambda i: (0, 0))
_row_shape = jax.ShapeDtypeStruct((NP, D), _f32)

_pre_call = pl.pallas_call(
    _tc_pre, grid=(GRID,),
    in_specs=[_row_spec, _w_spec, _hist_spec],
    out_specs=_row_spec, out_shape=_row_shape)

_mid_call = pl.pallas_call(
    _tc_mid, grid=(GRID,),
    in_specs=[_p_spec, _row_spec, _hist_spec, _w_spec, _b_spec],
    out_specs=_row_spec, out_shape=_row_shape)

_post_call = pl.pallas_call(
    _tc_post, grid=(GRID,),
    in_specs=[_p_spec, _row_spec, _hist_spec, _b_spec],
    out_specs=_row_spec, out_shape=_row_shape)


def kernel(x, edge_index, W1, b1, W2, b2):
    ei = edge_index.astype(jnp.int32)
    ei_t = ei.reshape(2, NW, E // NW)
    ei_p = jnp.pad(ei_t, ((0, 0), (0, 0), (0, EPT - E // NW)),
                   constant_values=NP - 1)
    src = ei_p[0].reshape(NC, NS, NB, K)
    dst = ei_p[1].reshape(NC, NS, NB, K)
    x_p = jnp.pad(x, ((0, NP - N), (0, 0)))
    b1r = b1.reshape(1, D)
    b2r = b2.reshape(1, D)

    hist = _deg_kernel(dst)
    g1 = _pre_call(x_p, W1, hist)
    p1 = _scatter_kernel(g1, src, dst)
    g2 = _mid_call(p1, g1, hist, W2, b1r)
    p2 = _scatter_kernel(g2, src, dst)
    out = _post_call(p2, g2, hist, b2r)
    return out[:N]


# speedup vs baseline: 1.7924x; 1.7924x over previous
import functools

import jax
import jax.numpy as jnp
from jax import lax
from jax.experimental import pallas as pl
from jax.experimental.pallas import tpu as pltpu
from jax.experimental.pallas import tpu_sc as plsc

N = 10000
NP = 10240
D = 128
E = 320000
NC = 2
NS = 16
NW = NC * NS
EPT = 10240
K = 128
NB = EPT // K
CH = 40
ROWS_PER_TILE = NP // NS
BLK = 1024
GRID = NP // BLK

_mesh = plsc.VectorSubcoreMesh(core_axis_name="c", subcore_axis_name="s")


@functools.partial(
    pl.kernel, mesh=_mesh,
    out_type=jax.ShapeDtypeStruct((NW, NP), jnp.float32),
    scratch_types=[
        pltpu.VMEM((NB, K), jnp.int32),
        pltpu.VMEM((NP,), jnp.float32),
    ],
    compiler_params=pltpu.CompilerParams(needs_layout_passes=False),
)
def _deg_kernel(dst_hbm, out_hbm, dst_v, hist_v):
    cid = lax.axis_index("c")
    sid = lax.axis_index("s")
    wid = cid * NS + sid
    pltpu.sync_copy(dst_hbm.at[cid, sid], dst_v)

    zv = jnp.zeros((16,), jnp.float32)

    def zbody(i, carry):
        hist_v[pl.ds(i * 16, 16)] = zv
        return carry

    lax.fori_loop(0, NP // 16, zbody, 0)

    ones = jnp.ones((16,), jnp.float32)

    def body(r, carry):
        for c in range(K // 16):
            idx = dst_v[r, pl.ds(c * 16, 16)]
            plsc.addupdate_scatter(hist_v, [idx], ones)
        return carry

    lax.fori_loop(0, NB, body, 0)
    pltpu.sync_copy(hist_v, out_hbm.at[wid])


@functools.partial(
    pl.kernel, mesh=_mesh,
    out_type=jax.ShapeDtypeStruct((NC, NP, D), jnp.float32),
    scratch_types=[
        pltpu.VMEM((CH, K), jnp.int32),
        pltpu.VMEM((CH, K), jnp.int32),
        pltpu.VMEM((K, D), jnp.float32),
        pltpu.VMEM((K, D), jnp.float32),
        pltpu.SemaphoreType.DMA,
        pltpu.SemaphoreType.DMA,
        pltpu.VMEM_SHARED((NP, D), jnp.float32),
    ],
    compiler_params=pltpu.CompilerParams(needs_layout_passes=False),
)
def _scatter_kernel(g_hbm, src_hbm, dst_hbm, out_hbm, src_v, dst_v, buf_a,
                    buf_b, sem_a, sem_b, acc_sh):
    cid = lax.axis_index("c")
    sid = lax.axis_index("s")

    zv = jnp.zeros((16,), jnp.float32)

    def zbody(r, carry):
        for c in range(D // 16):
            buf_a[r, pl.ds(c * 16, 16)] = zv
        return carry

    lax.fori_loop(0, K, zbody, 0)
    for c in range(ROWS_PER_TILE // K):
        pltpu.sync_copy(buf_a,
                        acc_sh.at[pl.ds(sid * ROWS_PER_TILE + c * K, K)])
    plsc.subcore_barrier()

    for ch in range(NB // CH):
        pltpu.sync_copy(src_hbm.at[cid, sid, pl.ds(ch * CH, CH)], src_v)
        pltpu.sync_copy(dst_hbm.at[cid, sid, pl.ds(ch * CH, CH)], dst_v)
        pltpu.async_copy(g_hbm.at[src_v.at[0]], buf_a, sem_a)

        def body(j, carry):
            e0 = j * 2
            e1 = e0 + 1
            pltpu.async_copy(g_hbm.at[src_v.at[e1]], buf_b, sem_b)
            pltpu.make_async_copy(g_hbm.at[src_v.at[e0]], buf_a, sem_a).wait()
            nxt = jnp.minimum(e0 + 2, CH - 1)
            pltpu.async_copy(g_hbm.at[src_v.at[nxt]], buf_a, sem_a)
            pltpu.make_async_copy(g_hbm.at[src_v.at[e1]], buf_b, sem_b).wait()
            return carry

        lax.fori_loop(0, CH // 2, body, 0)
        pltpu.make_async_copy(g_hbm.at[src_v.at[0]], buf_a, sem_a).wait()
    plsc.subcore_barrier()
    pltpu.sync_copy(acc_sh.at[pl.ds(sid * ROWS_PER_TILE, ROWS_PER_TILE)],
                    out_hbm.at[cid, pl.ds(sid * ROWS_PER_TILE, ROWS_PER_TILE)])


def _dinv_from(hist_ref):
    ones = jnp.ones((NW, D), jnp.float32)
    deg = lax.dot_general(hist_ref[...], ones, (((0,), (0,)), ((), ())),
                          preferred_element_type=jnp.float32)
    return lax.rsqrt(deg + 1.0)


def _tc_pre(x_ref, w_ref, hist_ref, g_ref):
    dinv = _dinv_from(hist_ref)
    g_ref[...] = dinv * jnp.dot(x_ref[...], w_ref[...],
                                preferred_element_type=jnp.float32)


def _tc_mid(p_ref, g_ref, hist_ref, w_ref, b_ref, o_ref):
    dinv = _dinv_from(hist_ref)
    s = p_ref[0] + p_ref[1] + g_ref[...]
    h = jnp.maximum(dinv * s + b_ref[...], 0.0)
    o_ref[...] = dinv * jnp.dot(h, w_ref[...],
                                preferred_element_type=jnp.float32)


def _tc_post(p_ref, g_ref, hist_ref, b_ref, o_ref):
    dinv = _dinv_from(hist_ref)
    s = p_ref[0] + p_ref[1] + g_ref[...]
    o_ref[...] = dinv * s + b_ref[...]


_f32 = jnp.float32
_hist_spec = pl.BlockSpec((NW, BLK), lambda i: (0, i))
_row_spec = pl.BlockSpec((BLK, D), lambda i: (i, 0))
_p_spec = pl.BlockSpec((NC, BLK, D), lambda i: (0, i, 0))
_w_spec = pl.BlockSpec((D, D), lambda i: (0, 0))
_b_spec = pl.BlockSpec((1, D), lambda i: (0, 0))
_row_shape = jax.ShapeDtypeStruct((NP, D), _f32)

_pre_call = pl.pallas_call(
    _tc_pre, grid=(GRID,),
    in_specs=[_row_spec, _w_spec, _hist_spec],
    out_specs=_row_spec, out_shape=_row_shape)

_mid_call = pl.pallas_call(
    _tc_mid, grid=(GRID,),
    in_specs=[_p_spec, _row_spec, _hist_spec, _w_spec, _b_spec],
    out_specs=_row_spec, out_shape=_row_shape)

_post_call = pl.pallas_call(
    _tc_post, grid=(GRID,),
    in_specs=[_p_spec, _row_spec, _hist_spec, _b_spec],
    out_specs=_row_spec, out_shape=_row_shape)


def kernel(x, edge_index, W1, b1, W2, b2):
    ei = edge_index.astype(jnp.int32)
    ei_t = ei.reshape(2, NW, E // NW)
    ei_p = jnp.pad(ei_t, ((0, 0), (0, 0), (0, EPT - E // NW)),
                   constant_values=NP - 1)
    src = ei_p[0].reshape(NC, NS, NB, K)
    dst = ei_p[1].reshape(NC, NS, NB, K)
    x_p = jnp.pad(x, ((0, NP - N), (0, 0)))
    b1r = b1.reshape(1, D)
    b2r = b2.reshape(1, D)

    hist = _deg_kernel(dst)
    g1 = _pre_call(x_p, W1, hist)
    p1 = _scatter_kernel(g1, src, dst)
    g2 = _mid_call(p1, g1, hist, W2, b1r)
    p2 = _scatter_kernel(g2, src, dst)
    out = _post_call(p2, g2, hist, b2r)
    return out[:N]


# speedup vs baseline: 2.1897x; 1.2217x over previous
import functools

import jax
import jax.numpy as jnp
from jax import lax
from jax.experimental import pallas as pl
from jax.experimental.pallas import tpu as pltpu
from jax.experimental.pallas import tpu_sc as plsc

N = 10000
NP = 10240
D = 128
E = 320000
NC = 2
NS = 16
NW = NC * NS
EPT = 10240
K = 128
NB = EPT // K
CH = 40
ROWS_PER_TILE = NP // NS
BLK = 1024
GRID = NP // BLK

_mesh = plsc.VectorSubcoreMesh(core_axis_name="c", subcore_axis_name="s")


@functools.partial(
    pl.kernel, mesh=_mesh,
    out_type=jax.ShapeDtypeStruct((NW, NP), jnp.float32),
    scratch_types=[
        pltpu.VMEM((NB, K), jnp.int32),
        pltpu.VMEM((NP,), jnp.float32),
    ],
    compiler_params=pltpu.CompilerParams(needs_layout_passes=False),
)
def _deg_kernel(dst_hbm, out_hbm, dst_v, hist_v):
    cid = lax.axis_index("c")
    sid = lax.axis_index("s")
    wid = cid * NS + sid
    pltpu.sync_copy(dst_hbm.at[cid, sid], dst_v)

    zv = jnp.zeros((16,), jnp.float32)

    def zbody(i, carry):
        hist_v[pl.ds(i * 16, 16)] = zv
        return carry

    lax.fori_loop(0, NP // 16, zbody, 0)

    ones = jnp.ones((16,), jnp.float32)

    def body(r, carry):
        for c in range(K // 16):
            idx = dst_v[r, pl.ds(c * 16, 16)]
            plsc.addupdate_scatter(hist_v, [idx], ones)
        return carry

    lax.fori_loop(0, NB, body, 0)
    pltpu.sync_copy(hist_v, out_hbm.at[wid])


@functools.partial(
    pl.kernel, mesh=_mesh,
    out_type=jax.ShapeDtypeStruct((NC, NP, D), jnp.float32),
    scratch_types=[
        pltpu.VMEM((CH, K), jnp.int32),
        pltpu.VMEM((CH, K), jnp.int32),
        pltpu.VMEM((K, D), jnp.float32),
        pltpu.VMEM((K, D), jnp.float32),
        pltpu.SemaphoreType.DMA,
        pltpu.SemaphoreType.DMA,
        pltpu.SemaphoreType.DMA,
        pltpu.SemaphoreType.DMA,
        pltpu.VMEM_SHARED((NP, D), jnp.float32),
    ],
    compiler_params=pltpu.CompilerParams(needs_layout_passes=False),
)
def _scatter_kernel(g_hbm, src_hbm, dst_hbm, out_hbm, src_v, dst_v, buf_a,
                    buf_b, sem_a, sem_b, sem_c, sem_d, acc_sh):
    cid = lax.axis_index("c")
    sid = lax.axis_index("s")

    zv = jnp.zeros((16,), jnp.float32)

    def zbody(r, carry):
        for c in range(D // 16):
            buf_a[r, pl.ds(c * 16, 16)] = zv
        return carry

    lax.fori_loop(0, K, zbody, 0)
    for c in range(ROWS_PER_TILE // K):
        pltpu.sync_copy(buf_a,
                        acc_sh.at[pl.ds(sid * ROWS_PER_TILE + c * K, K)])
    plsc.subcore_barrier()

    for ch in range(NB // CH):
        pltpu.sync_copy(src_hbm.at[cid, sid, pl.ds(ch * CH, CH)], src_v)
        pltpu.sync_copy(dst_hbm.at[cid, sid, pl.ds(ch * CH, CH)], dst_v)

        def body(j, carry):
            e0 = j * 2
            e1 = e0 + 1
            ha = pltpu.async_copy(g_hbm.at[src_v.at[e0]], buf_a, sem_a)
            hb = pltpu.async_copy(g_hbm.at[src_v.at[e1]], buf_b, sem_b)
            ha.wait()
            sa = pltpu.async_copy(buf_a, acc_sh.at[dst_v.at[e0]], sem_c,
                                  add=True)
            hb.wait()
            sb = pltpu.async_copy(buf_b, acc_sh.at[dst_v.at[e1]], sem_d,
                                  add=True)
            sa.wait()
            sb.wait()
            return carry

        lax.fori_loop(0, CH // 2, body, 0)
    plsc.subcore_barrier()
    pltpu.sync_copy(acc_sh.at[pl.ds(sid * ROWS_PER_TILE, ROWS_PER_TILE)],
                    out_hbm.at[cid, pl.ds(sid * ROWS_PER_TILE, ROWS_PER_TILE)])


def _dinv_from(hist_ref):
    ones = jnp.ones((NW, D), jnp.float32)
    deg = lax.dot_general(hist_ref[...], ones, (((0,), (0,)), ((), ())),
                          preferred_element_type=jnp.float32)
    return lax.rsqrt(deg + 1.0)


def _tc_pre(x_ref, w_ref, hist_ref, g_ref):
    dinv = _dinv_from(hist_ref)
    g_ref[...] = dinv * jnp.dot(x_ref[...], w_ref[...],
                                preferred_element_type=jnp.float32)


def _tc_mid(p_ref, g_ref, hist_ref, w_ref, b_ref, o_ref):
    dinv = _dinv_from(hist_ref)
    s = p_ref[0] + p_ref[1] + g_ref[...]
    h = jnp.maximum(dinv * s + b_ref[...], 0.0)
    o_ref[...] = dinv * jnp.dot(h, w_ref[...],
                                preferred_element_type=jnp.float32)


def _tc_post(p_ref, g_ref, hist_ref, b_ref, o_ref):
    dinv = _dinv_from(hist_ref)
    s = p_ref[0] + p_ref[1] + g_ref[...]
    o_ref[...] = dinv * s + b_ref[...]


_f32 = jnp.float32
_hist_spec = pl.BlockSpec((NW, BLK), lambda i: (0, i))
_row_spec = pl.BlockSpec((BLK, D), lambda i: (i, 0))
_p_spec = pl.BlockSpec((NC, BLK, D), lambda i: (0, i, 0))
_w_spec = pl.BlockSpec((D, D), lambda i: (0, 0))
_b_spec = pl.BlockSpec((1, D), lambda i: (0, 0))
_row_shape = jax.ShapeDtypeStruct((NP, D), _f32)

_pre_call = pl.pallas_call(
    _tc_pre, grid=(GRID,),
    in_specs=[_row_spec, _w_spec, _hist_spec],
    out_specs=_row_spec, out_shape=_row_shape)

_mid_call = pl.pallas_call(
    _tc_mid, grid=(GRID,),
    in_specs=[_p_spec, _row_spec, _hist_spec, _w_spec, _b_spec],
    out_specs=_row_spec, out_shape=_row_shape)

_post_call = pl.pallas_call(
    _tc_post, grid=(GRID,),
    in_specs=[_p_spec, _row_spec, _hist_spec, _b_spec],
    out_specs=_row_spec, out_shape=_row_shape)


def kernel(x, edge_index, W1, b1, W2, b2):
    ei = edge_index.astype(jnp.int32)
    ei_t = ei.reshape(2, NW, E // NW)
    ei_p = jnp.pad(ei_t, ((0, 0), (0, 0), (0, EPT - E // NW)),
                   constant_values=NP - 1)
    src = ei_p[0].reshape(NC, NS, NB, K)
    dst = ei_p[1].reshape(NC, NS, NB, K)
    x_p = jnp.pad(x, ((0, NP - N), (0, 0)))
    b1r = b1.reshape(1, D)
    b2r = b2.reshape(1, D)

    hist = _deg_kernel(dst)
    g1 = _pre_call(x_p, W1, hist)
    p1 = _scatter_kernel(g1, src, dst)
    g2 = _mid_call(p1, g1, hist, W2, b1r)
    p2 = _scatter_kernel(g2, src, dst)
    out = _post_call(p2, g2, hist, b2r)
    return out[:N]
